# Initial kernel scaffold; baseline (speedup 1.0000x reference)
import functools

import jax
import jax.numpy as jnp
from jax import lax
from jax.experimental import pallas as pl
from jax.experimental.pallas import tpu as pltpu
from jax.experimental.pallas import tpu_sc as plsc

_CHUNK = 1024


def _gather_body(b_per_w, n_chunks, num_cores,
                 table_hbm, idx_hbm, out_hbm, idx_v, rows_v, sem):
    wid = lax.axis_index("s") * num_cores + lax.axis_index("c")
    base = wid * b_per_w

    @pl.loop(0, n_chunks)
    def _step(i):
        off = base + i * _CHUNK
        pltpu.sync_copy(idx_hbm.at[pl.ds(off, _CHUNK)], idx_v)
        pltpu.async_copy(table_hbm.at[idx_v], rows_v, sem).wait()
        pltpu.sync_copy(rows_v, out_hbm.at[pl.ds(off, _CHUNK)])


def kernel(input, weight):
    n, s = input.shape
    num_rows, dim = weight.shape
    b = n * s
    idx_flat = input.reshape(b).astype(jnp.int32)

    info = plsc.get_sparse_core_info()
    num_workers = info.num_cores * info.num_subcores
    b_per_w = b // num_workers
    n_chunks = b_per_w // _CHUNK

    mesh = plsc.VectorSubcoreMesh(core_axis_name="c", subcore_axis_name="s")
    gather = pl.kernel(
        functools.partial(_gather_body, b_per_w, n_chunks, info.num_cores),
        out_type=jax.ShapeDtypeStruct((b, dim), jnp.float32),
        mesh=mesh,
        scratch_types=[
            pltpu.VMEM((_CHUNK,), jnp.int32),
            pltpu.VMEM((_CHUNK, dim), jnp.float32),
            pltpu.SemaphoreType.DMA,
        ],
    )
    out = gather(weight, idx_flat)
    return out.reshape(n, s, dim)


# speedup vs baseline: 1.0946x; 1.0946x over previous
import functools

import jax
import jax.numpy as jnp
from jax import lax
from jax.experimental import pallas as pl
from jax.experimental.pallas import tpu as pltpu
from jax.experimental.pallas import tpu_sc as plsc

_CHUNK = 1024


def _gather_body(b_per_w, n_chunks, num_cores,
                 table_hbm, idx_hbm, out_hbm, idx_v, rows_v, sem):
    wid = lax.axis_index("s") * num_cores + lax.axis_index("c")
    base = wid * b_per_w

    @pl.loop(0, n_chunks)
    def _step(i):
        off = base + i * _CHUNK
        pltpu.sync_copy(idx_hbm.at[pl.ds(off, _CHUNK)], idx_v)
        pltpu.async_copy(table_hbm.at[idx_v], rows_v, sem).wait()
        pltpu.sync_copy(rows_v, out_hbm.at[pl.ds(off, _CHUNK)])


def kernel(input, weight):
    n, s = input.shape
    num_rows, dim = weight.shape
    b = n * s
    idx_flat = input.reshape(b).astype(jnp.int32)

    info = plsc.get_sparse_core_info()
    num_workers = info.num_cores * info.num_subcores
    b_per_w = b // num_workers
    n_chunks = b_per_w // _CHUNK

    mesh = plsc.VectorSubcoreMesh(core_axis_name="c", subcore_axis_name="s")
    gather = pl.kernel(
        functools.partial(_gather_body, b_per_w, n_chunks, info.num_cores),
        out_type=jax.ShapeDtypeStruct((b, dim), jnp.float32),
        mesh=mesh,
        scratch_types=[
            pltpu.VMEM((_CHUNK,), jnp.int32),
            pltpu.VMEM((_CHUNK, dim), jnp.float32),
            pltpu.SemaphoreType.DMA,
        ],
        compiler_params=pltpu.CompilerParams(use_tc_tiling_on_sc=False),
    )
    out = gather(weight, idx_flat)
    return out.reshape(n, s, dim)


# speedup vs baseline: 1.1127x; 1.0166x over previous
import functools

import jax
import jax.numpy as jnp
from jax import lax
from jax.experimental import pallas as pl
from jax.experimental.pallas import tpu as pltpu
from jax.experimental.pallas import tpu_sc as plsc

_CHUNK = 800
_NBUF = 4


def _gather_body(b_per_w, n_chunks, num_cores,
                 table_hbm, idx_hbm, out_hbm, idx_v, rows, gsems, ssems):
    wid = lax.axis_index("s") * num_cores + lax.axis_index("c")
    base = wid * b_per_w

    pltpu.sync_copy(idx_hbm.at[pl.ds(base, b_per_w)], idx_v)

    def gather_start(c, b):
        pltpu.async_copy(
            table_hbm.at[idx_v.at[pl.ds(c * _CHUNK, _CHUNK)]], rows[b], gsems[b])

    def gather_wait(b):
        pltpu.make_async_copy(
            table_hbm.at[idx_v.at[pl.ds(0, _CHUNK)]], rows[b], gsems[b]).wait()

    def store_start(c, b):
        pltpu.async_copy(rows[b], out_hbm.at[pl.ds(base + c * _CHUNK, _CHUNK)],
                         ssems[b])

    def store_wait(b):
        pltpu.make_async_copy(
            rows[b], out_hbm.at[pl.ds(0, _CHUNK)], ssems[b]).wait()

    for b in range(_NBUF):
        gather_start(b, b)

    @pl.loop(0, n_chunks // _NBUF)
    def _outer(o):
        for b in range(_NBUF):
            c = o * _NBUF + b
            gather_wait(b)
            store_start(c, b)
            cn = c + _NBUF

            @pl.when(cn < n_chunks)
            def _():
                store_wait(b)
                gather_start(cn, b)

    for b in range(_NBUF):
        store_wait(b)


def kernel(input, weight):
    n, s = input.shape
    num_rows, dim = weight.shape
    b = n * s
    idx_flat = input.reshape(b).astype(jnp.int32)

    info = plsc.get_sparse_core_info()
    num_workers = info.num_cores * info.num_subcores
    b_per_w = b // num_workers
    n_chunks = b_per_w // _CHUNK

    mesh = plsc.VectorSubcoreMesh(core_axis_name="c", subcore_axis_name="s")
    gather = pl.kernel(
        functools.partial(_gather_body, b_per_w, n_chunks, info.num_cores),
        out_type=jax.ShapeDtypeStruct((b, dim), jnp.float32),
        mesh=mesh,
        scratch_types=[
            pltpu.VMEM((b_per_w,), jnp.int32),
            [pltpu.VMEM((_CHUNK, dim), jnp.float32) for _ in range(_NBUF)],
            [pltpu.SemaphoreType.DMA for _ in range(_NBUF)],
            [pltpu.SemaphoreType.DMA for _ in range(_NBUF)],
        ],
        compiler_params=pltpu.CompilerParams(use_tc_tiling_on_sc=False),
    )
    out = gather(weight, idx_flat)
    return out.reshape(n, s, dim)


# speedup vs baseline: 1.7975x; 1.6155x over previous
import functools

import jax
import jax.numpy as jnp
from jax import lax
from jax.experimental import pallas as pl
from jax.experimental.pallas import tpu as pltpu
from jax.experimental.pallas import tpu_sc as plsc

_NBUF = 8


def _gather_body(rows_per_w, num_cores,
                 table_hbm, idx_hbm, out_hbm, idx_v, rows, gsems, ssems):
    wid = lax.axis_index("s") * num_cores + lax.axis_index("c")
    base = wid * rows_per_w

    pltpu.sync_copy(idx_hbm.at[pl.ds(base, rows_per_w), :], idx_v)

    def gather_start(c, b):
        pltpu.async_copy(table_hbm.at[idx_v.at[c]], rows[b], gsems[b])

    def gather_wait(b):
        pltpu.make_async_copy(table_hbm.at[idx_v.at[0]], rows[b],
                              gsems[b]).wait()

    def store_start(c, b):
        pltpu.async_copy(rows[b], out_hbm.at[base + c], ssems[b])

    def store_wait(b):
        pltpu.make_async_copy(rows[b], out_hbm.at[0], ssems[b]).wait()

    for b in range(_NBUF):
        gather_start(b, b)

    @pl.loop(0, rows_per_w // _NBUF)
    def _outer(o):
        for b in range(_NBUF):
            c = o * _NBUF + b
            gather_wait(b)
            store_start(c, b)
            cn = c + _NBUF

            @pl.when(cn < rows_per_w)
            def _():
                store_wait(b)
                gather_start(cn, b)

    for b in range(_NBUF):
        store_wait(b)


def kernel(input, weight):
    n, s = input.shape
    num_rows, dim = weight.shape
    idx = input.astype(jnp.int32)

    info = plsc.get_sparse_core_info()
    num_workers = info.num_cores * info.num_subcores
    rows_per_w = n // num_workers

    mesh = plsc.VectorSubcoreMesh(core_axis_name="c", subcore_axis_name="s")
    gather = pl.kernel(
        functools.partial(_gather_body, rows_per_w, info.num_cores),
        out_type=jax.ShapeDtypeStruct((n, s, dim), jnp.float32),
        mesh=mesh,
        scratch_types=[
            pltpu.VMEM((rows_per_w, s), jnp.int32),
            [pltpu.VMEM((s, dim), jnp.float32) for _ in range(_NBUF)],
            [pltpu.SemaphoreType.DMA for _ in range(_NBUF)],
            [pltpu.SemaphoreType.DMA for _ in range(_NBUF)],
        ],
        compiler_params=pltpu.CompilerParams(use_tc_tiling_on_sc=False),
    )
    return gather(weight, idx)
